# Initial kernel scaffold; baseline (speedup 1.0000x reference)
import jax
import jax.numpy as jnp
from jax.experimental import pallas as pl


def kernel(x, edge_index, W1, b1, W2, b2, W3, b3):
    raise NotImplementedError("write your pallas kernel here")


# speedup vs baseline: 2.8997x; 2.8997x over previous
import functools

import jax
import jax.numpy as jnp
from jax import lax
from jax.experimental import pallas as pl
from jax.experimental.pallas import tpu as pltpu
from jax.experimental.pallas import tpu_sc as plsc

NN = 10000
DD = 128
EE = 320000

NTILES = 32
CK = 128
CHUNKS = 80
E_PAD = NTILES * CHUNKS * CK
N_PAD = 10240
RPT = N_PAD // 16
DUMMY = NN

_mesh = plsc.VectorSubcoreMesh(core_axis_name="c", subcore_axis_name="s")


@functools.partial(
    pl.kernel,
    out_type=jax.ShapeDtypeStruct((2, N_PAD, DD), jnp.float32),
    mesh=_mesh,
    scratch_types=[
        pltpu.VMEM_SHARED((N_PAD, DD), jnp.float32),
        pltpu.VMEM((CHUNKS, CK), jnp.int32),
        pltpu.VMEM((CHUNKS, CK), jnp.int32),
        pltpu.VMEM((CK, DD), jnp.float32),
        pltpu.SemaphoreType.DMA,
    ],
)
def _sc_agg(h_hbm, srcs_hbm, dsts_hbm, out_hbm, agg_sh, sidx, didx, rows, sem):
    c = lax.axis_index("c")
    s = lax.axis_index("s")
    wid = c * 16 + s
    pltpu.sync_copy(srcs_hbm.at[wid], sidx)
    pltpu.sync_copy(dsts_hbm.at[wid], didx)
    pltpu.sync_copy(
        h_hbm.at[pl.ds(s * RPT, RPT)],
        agg_sh.at[pl.ds(s * RPT, RPT)],
    )
    plsc.subcore_barrier()

    def chunk(j, carry):
        pltpu.async_copy(h_hbm.at[sidx.at[j]], rows, sem).wait()
        pltpu.sync_copy(rows, agg_sh.at[didx.at[j]], add=True)
        return carry

    lax.fori_loop(0, CHUNKS, chunk, 0)
    plsc.subcore_barrier()
    pltpu.sync_copy(
        agg_sh.at[pl.ds(s * RPT, RPT)],
        out_hbm.at[c, pl.ds(s * RPT, RPT)],
    )


_BM = 640


def _mm_body(h_ref, p_ref, w_ref, b_ref, o_ref):
    rst = p_ref[0] + p_ref[1] - h_ref[...]
    o_ref[...] = (
        jnp.dot(rst, w_ref[...], preferred_element_type=jnp.float32) + b_ref[...]
    )


def _tc_mm(h, parts, w, b):
    return pl.pallas_call(
        _mm_body,
        grid=(N_PAD // _BM,),
        in_specs=[
            pl.BlockSpec((_BM, DD), lambda i: (i, 0)),
            pl.BlockSpec((2, _BM, DD), lambda i: (0, i, 0)),
            pl.BlockSpec((DD, DD), lambda i: (0, 0)),
            pl.BlockSpec((1, DD), lambda i: (0, 0)),
        ],
        out_specs=pl.BlockSpec((_BM, DD), lambda i: (i, 0)),
        out_shape=jax.ShapeDtypeStruct((N_PAD, DD), jnp.float32),
    )(h, parts, w, b.reshape(1, DD))


def kernel(x, edge_index, W1, b1, W2, b2, W3, b3):
    pad = E_PAD - EE
    src = jnp.concatenate([edge_index[0], jnp.zeros((pad,), jnp.int32)])
    dst = jnp.concatenate([edge_index[1], jnp.full((pad,), DUMMY, jnp.int32)])
    srcs = src.reshape(NTILES, CHUNKS, CK)
    dsts = dst.reshape(NTILES, CHUNKS, CK)

    h = jnp.pad(x, ((0, N_PAD - NN), (0, 0)))
    for w, b in ((W1, b1), (W2, b2), (W3, b3)):
        parts = _sc_agg(h, srcs, dsts)
        h = _tc_mm(h, parts, w, b)
    return h[:NN]


# speedup vs baseline: 3.1758x; 1.0952x over previous
import functools

import jax
import jax.numpy as jnp
from jax import lax
from jax.experimental import pallas as pl
from jax.experimental.pallas import tpu as pltpu
from jax.experimental.pallas import tpu_sc as plsc

NN = 10000
DD = 128
EE = 320000

NTILES = 32
CK = 128
CHUNKS = 80
E_PAD = NTILES * CHUNKS * CK
N_PAD = 10240
RPT = N_PAD // 16
DUMMY = NN

_mesh = plsc.VectorSubcoreMesh(core_axis_name="c", subcore_axis_name="s")


@functools.partial(
    pl.kernel,
    out_type=jax.ShapeDtypeStruct((2, N_PAD, DD), jnp.float32),
    mesh=_mesh,
    scratch_types=[
        pltpu.VMEM_SHARED((N_PAD, DD), jnp.float32),
        pltpu.VMEM((CHUNKS, CK), jnp.int32),
        pltpu.VMEM((1, CK), jnp.int32),
        pltpu.VMEM((1, CK), jnp.int32),
        pltpu.VMEM((CK, DD), jnp.float32),
        pltpu.VMEM((CK, DD), jnp.float32),
        pltpu.SemaphoreType.DMA,
        pltpu.SemaphoreType.DMA,
        pltpu.SemaphoreType.DMA,
        pltpu.SemaphoreType.DMA,
    ],
)
def _sc_agg(h_hbm, srcs_hbm, dsts_hbm, out_hbm, agg_sh, didx, ib0, ib1,
            rows0, rows1, gsem0, gsem1, isem0, isem1):
    c = lax.axis_index("c")
    s = lax.axis_index("s")
    wid = c * 16 + s
    pltpu.sync_copy(dsts_hbm.at[wid], didx)
    pltpu.sync_copy(
        h_hbm.at[pl.ds(s * RPT, RPT)],
        agg_sh.at[pl.ds(s * RPT, RPT)],
    )
    plsc.subcore_barrier()

    ibs = (ib0, ib1)
    isems = (isem0, isem1)
    rows = (rows0, rows1)
    gsems = (gsem0, gsem1)
    pltpu.async_copy(srcs_hbm.at[wid, 0], ib0, isem0)
    pltpu.async_copy(srcs_hbm.at[wid, 1], ib1, isem1)
    pltpu.make_async_copy(srcs_hbm.at[wid, 0], ib0, isem0).wait()
    pltpu.async_copy(h_hbm.at[ib0.at[0]], rows0, gsem0)

    def pair(i, carry):
        j0 = 2 * i
        for b in range(2):
            j = j0 + b
            pltpu.make_async_copy(h_hbm.at[ibs[b].at[0]], rows[b], gsems[b]).wait()

            @pl.when(j + 2 < CHUNKS)
            def _():
                pltpu.async_copy(srcs_hbm.at[wid, j + 2], ibs[b], isems[b])

            @pl.when(j + 1 < CHUNKS)
            def _():
                pltpu.make_async_copy(
                    srcs_hbm.at[wid, 0], ibs[1 - b], isems[1 - b]
                ).wait()
                pltpu.async_copy(h_hbm.at[ibs[1 - b].at[0]], rows[1 - b],
                                 gsems[1 - b])

            pltpu.sync_copy(rows[b], agg_sh.at[didx.at[j]], add=True)
        return carry

    lax.fori_loop(0, CHUNKS // 2, pair, 0)
    plsc.subcore_barrier()
    pltpu.sync_copy(
        agg_sh.at[pl.ds(s * RPT, RPT)],
        out_hbm.at[c, pl.ds(s * RPT, RPT)],
    )


_BM = 640


def _mm_body(h_ref, p_ref, w_ref, b_ref, o_ref):
    rst = p_ref[0] + p_ref[1] - h_ref[...]
    o_ref[...] = (
        jnp.dot(rst, w_ref[...], preferred_element_type=jnp.float32) + b_ref[...]
    )


def _tc_mm(h, parts, w, b):
    return pl.pallas_call(
        _mm_body,
        grid=(N_PAD // _BM,),
        in_specs=[
            pl.BlockSpec((_BM, DD), lambda i: (i, 0)),
            pl.BlockSpec((2, _BM, DD), lambda i: (0, i, 0)),
            pl.BlockSpec((DD, DD), lambda i: (0, 0)),
            pl.BlockSpec((1, DD), lambda i: (0, 0)),
        ],
        out_specs=pl.BlockSpec((_BM, DD), lambda i: (i, 0)),
        out_shape=jax.ShapeDtypeStruct((N_PAD, DD), jnp.float32),
    )(h, parts, w, b.reshape(1, DD))


def kernel(x, edge_index, W1, b1, W2, b2, W3, b3):
    pad = E_PAD - EE
    src = jnp.concatenate([edge_index[0], jnp.zeros((pad,), jnp.int32)])
    dst = jnp.concatenate([edge_index[1], jnp.full((pad,), DUMMY, jnp.int32)])
    srcs = src.reshape(NTILES, CHUNKS, 1, CK)
    dsts = dst.reshape(NTILES, CHUNKS, CK)

    h = jnp.pad(x, ((0, N_PAD - NN), (0, 0)))
    for w, b in ((W1, b1), (W2, b2), (W3, b3)):
        parts = _sc_agg(h, srcs, dsts)
        h = _tc_mm(h, parts, w, b)
    return h[:NN]


# speedup vs baseline: 9.5947x; 3.0212x over previous
import functools

import jax
import jax.numpy as jnp
from jax import lax
from jax.experimental import pallas as pl
from jax.experimental.pallas import tpu as pltpu
from jax.experimental.pallas import tpu_sc as plsc

NN = 10000
DD = 128
EE = 320000

NTILES = 32
CK = 128
CHUNKS = 80
E_PAD = NTILES * CHUNKS * CK
N_PAD = 10240
RPT = N_PAD // 16
DUMMY = NN

_mesh = plsc.VectorSubcoreMesh(core_axis_name="c", subcore_axis_name="s")


@functools.partial(
    pl.kernel,
    out_type=jax.ShapeDtypeStruct((2, N_PAD, DD), jnp.float32),
    mesh=_mesh,
    scratch_types=[
        pltpu.VMEM_SHARED((N_PAD, DD), jnp.float32),
        pltpu.VMEM((CHUNKS, CK), jnp.int32),
        pltpu.VMEM((1, CK), jnp.int32),
        pltpu.VMEM((1, CK), jnp.int32),
        pltpu.VMEM((CK, DD), jnp.float32),
        pltpu.VMEM((CK, DD), jnp.float32),
        pltpu.SemaphoreType.DMA,
        pltpu.SemaphoreType.DMA,
        pltpu.SemaphoreType.DMA,
        pltpu.SemaphoreType.DMA,
    ],
)
def _sc_agg(h_hbm, srcs_hbm, dsts_hbm, out_hbm, agg_sh, didx, ib0, ib1,
            rows0, rows1, gsem0, gsem1, isem0, isem1):
    c = lax.axis_index("c")
    s = lax.axis_index("s")
    wid = c * 16 + s
    pltpu.sync_copy(dsts_hbm.at[wid], didx)
    pltpu.sync_copy(
        h_hbm.at[pl.ds(s * RPT, RPT)],
        agg_sh.at[pl.ds(s * RPT, RPT)],
    )
    plsc.subcore_barrier()

    ibs = (ib0, ib1)
    isems = (isem0, isem1)
    rows = (rows0, rows1)
    gsems = (gsem0, gsem1)
    pltpu.async_copy(srcs_hbm.at[wid, 0], ib0, isem0)
    pltpu.async_copy(srcs_hbm.at[wid, 1], ib1, isem1)
    pltpu.make_async_copy(srcs_hbm.at[wid, 0], ib0, isem0).wait()
    pltpu.async_copy(h_hbm.at[ib0.at[0]], rows0, gsem0)

    def pair(i, carry):
        j0 = 2 * i
        for b in range(2):
            j = j0 + b
            pltpu.make_async_copy(h_hbm.at[ibs[b].at[0]], rows[b], gsems[b]).wait()

            @pl.when(j + 2 < CHUNKS)
            def _():
                pltpu.async_copy(srcs_hbm.at[wid, j + 2], ibs[b], isems[b])

            @pl.when(j + 1 < CHUNKS)
            def _():
                pltpu.make_async_copy(
                    srcs_hbm.at[wid, 0], ibs[1 - b], isems[1 - b]
                ).wait()
                pltpu.async_copy(h_hbm.at[ibs[1 - b].at[0]], rows[1 - b],
                                 gsems[1 - b])

            pltpu.sync_copy(rows[b], agg_sh.at[didx.at[j]], add=True)
        return carry

    lax.fori_loop(0, CHUNKS // 2, pair, 0)
    plsc.subcore_barrier()
    pltpu.sync_copy(
        agg_sh.at[pl.ds(s * RPT, RPT)],
        out_hbm.at[c, pl.ds(s * RPT, RPT)],
    )


_BM = 640


def _mm_body(h_ref, p_ref, w_ref, b_ref, o_ref):
    rst = p_ref[0] + p_ref[1] - h_ref[...]
    o_ref[...] = (
        jnp.dot(rst, w_ref[...], preferred_element_type=jnp.float32) + b_ref[...]
    )


def _tc_mm(h, parts, w, b):
    return pl.pallas_call(
        _mm_body,
        grid=(N_PAD // _BM,),
        in_specs=[
            pl.BlockSpec((_BM, DD), lambda i: (i, 0)),
            pl.BlockSpec((2, _BM, DD), lambda i: (0, i, 0)),
            pl.BlockSpec((DD, DD), lambda i: (0, 0)),
            pl.BlockSpec((1, DD), lambda i: (0, 0)),
        ],
        out_specs=pl.BlockSpec((_BM, DD), lambda i: (i, 0)),
        out_shape=jax.ShapeDtypeStruct((N_PAD, DD), jnp.float32),
    )(h, parts, w, b.reshape(1, DD))


def kernel(x, edge_index, W1, b1, W2, b2, W3, b3):
    pad = E_PAD - EE
    pad_src = jnp.arange(pad, dtype=jnp.int32) % NN
    pad_dst = DUMMY + jnp.arange(pad, dtype=jnp.int32) % (N_PAD - NN)
    src = jnp.concatenate([edge_index[0], pad_src])
    dst = jnp.concatenate([edge_index[1], pad_dst])
    srcs = src.reshape(NTILES, CHUNKS, 1, CK)
    dsts = dst.reshape(NTILES, CHUNKS, CK)

    h = jnp.pad(x, ((0, N_PAD - NN), (0, 0)))
    for w, b in ((W1, b1), (W2, b2), (W3, b3)):
        parts = _sc_agg(h, srcs, dsts)
        h = _tc_mm(h, parts, w, b)
    return h[:NN]
